# Initial kernel scaffold; baseline (speedup 1.0000x reference)
import jax
import jax.numpy as jnp
from jax.experimental import pallas as pl


def kernel(probs, attn, targ, align, src, eps):
    raise NotImplementedError("write your pallas kernel here")


# speedup vs baseline: 4.7348x; 4.7348x over previous
import jax
import jax.numpy as jnp
from jax import lax
from jax.experimental import pallas as pl
from jax.experimental.pallas import tpu as pltpu
from jax.experimental.pallas import tpu_sc as plsc

N = 2048
V = 32000
S = 512
L = 16
NW = 32
RPW = N // NW
BIG = 1 << 30


def _merge_argmax(va, pa, vb, pb):
    v = jnp.maximum(va, vb)
    p = jnp.where(va > vb, pa, jnp.where(vb > va, pb, jnp.minimum(pa, pb)))
    return v, p


def _sc_body(probs, attn, align, src, targ, skey, sval,
             pred_out, tgen_out, scop_out,
             pbuf0, pbuf1, attnv, alignv, srcv, skv, svv,
             targv, predv, tgenv, scopv, sem0, sem1):
    cc = lax.axis_index("c")
    ss = lax.axis_index("s")
    wid = ss * 2 + cc
    base = wid * RPW
    iota = lax.iota(jnp.int32, L)
    zf = jnp.zeros((L,), jnp.float32)
    neg = jnp.full((L,), jnp.float32(-3.0e38))
    zi = jnp.zeros((L,), jnp.int32)

    pltpu.sync_copy(targ.at[pl.ds(base, RPW)], targv)
    skv[pl.ds(S, L)] = jnp.full((L,), jnp.int32(-1))
    pltpu.async_copy(probs.at[base], pbuf0, sem0)

    def row_body(rl, carry):
        r = base + rl
        pltpu.sync_copy(attn.at[r], attnv)
        pltpu.sync_copy(align.at[r], alignv)
        pltpu.sync_copy(src.at[r], srcv)
        pltpu.sync_copy(skey.at[pl.ds(r * S, S)], skv.at[pl.ds(0, S)])
        pltpu.sync_copy(sval.at[pl.ds(r * S, S)], svv)

        def cbody(k, ca):
            csum, cmax, cidx = ca
            cp = attnv[pl.ds(k * L, L)] * alignv[pl.ds(k * L, L)]
            m = cp > cmax
            return (csum + cp, jnp.maximum(cmax, cp),
                    jnp.where(m, jnp.full((L,), k), cidx))
        csum, cmax, cidx = lax.fori_loop(0, S // L, cbody, (zf, neg, zi))
        scop = jnp.sum(csum)
        mcop = jnp.max(cmax)
        cpos = cidx * L + iota
        icop = jnp.min(jnp.where(cmax == mcop, cpos, BIG))
        oovv = plsc.load_gather(srcv, [jnp.full((L,), icop)])

        def dense_and_cand(pb):
            def dbody(i, da):
                m0, m1, m2, m3, i0, i1, i2, i3 = da
                isp = jnp.full((L,), i)
                p = pb[pl.ds(i * 64, L)]
                k = p > m0
                m0 = jnp.maximum(m0, p); i0 = jnp.where(k, isp, i0)
                p = pb[pl.ds(i * 64 + 16, L)]
                k = p > m1
                m1 = jnp.maximum(m1, p); i1 = jnp.where(k, isp, i1)
                p = pb[pl.ds(i * 64 + 32, L)]
                k = p > m2
                m2 = jnp.maximum(m2, p); i2 = jnp.where(k, isp, i2)
                p = pb[pl.ds(i * 64 + 48, L)]
                k = p > m3
                m3 = jnp.maximum(m3, p); i3 = jnp.where(k, isp, i3)
                return m0, m1, m2, m3, i0, i1, i2, i3
            m0, m1, m2, m3, i0, i1, i2, i3 = lax.fori_loop(
                0, V // 64, dbody, (neg, neg, neg, neg, zi, zi, zi, zi))
            va, pa = _merge_argmax(m0, i0 * 64 + iota, m1, i1 * 64 + 16 + iota)
            vb, pb2 = _merge_argmax(m2, i2 * 64 + 32 + iota,
                                    m3, i3 * 64 + 48 + iota)
            mv, mp = _merge_argmax(va, pa, vb, pb2)
            dmax = jnp.max(mv)
            didx = jnp.min(jnp.where(mv == dmax, mp, BIG))
            tridx = plsc.load_gather(targv, [jnp.full((L,), rl)])
            tg = plsc.load_gather(pb, [tridx])

            rbase = r * V

            def gbody(k, ga):
                gmax, gidx = ga
                kk = skv[pl.ds(k * L, L)]
                kn = skv[pl.ds(k * L + 1, L)]
                vv = svv[pl.ds(k * L, L)]
                slot = kk - rbase
                gp = plsc.load_gather(pb, [slot])
                cand = jnp.where(kk != kn, gp + vv, neg)
                m = cand > gmax
                return jnp.maximum(gmax, cand), jnp.where(m, slot, gidx)
            gmax, gidx = lax.fori_loop(0, S // L, gbody, (neg, zi))
            candmax = jnp.max(gmax)
            candidx = jnp.min(jnp.where(gmax == candmax, gidx, BIG))
            return dmax, didx, tg, candmax, candidx

        even = (rl % 2) == 0

        def with_buf0(_):
            pltpu.make_async_copy(probs.at[base], pbuf0, sem0).wait()
            @pl.when(rl < RPW - 1)
            def _():
                pltpu.async_copy(probs.at[r + 1], pbuf1, sem1)
            return dense_and_cand(pbuf0)

        def with_buf1(_):
            pltpu.make_async_copy(probs.at[base], pbuf1, sem1).wait()
            @pl.when(rl < RPW - 1)
            def _():
                pltpu.async_copy(probs.at[r + 1], pbuf0, sem0)
            return dense_and_cand(pbuf1)

        dmax, didx, tg, candmax, candidx = lax.cond(
            even, with_buf0, with_buf1, 0)

        mvoc = jnp.maximum(dmax, candmax)
        ivoc = jnp.where(candmax > dmax, candidx, didx)
        predvec = jnp.where(mvoc < mcop, oovv, jnp.full((L,), ivoc))
        rlv = jnp.full((L,), rl)
        lane0 = iota == 0
        plsc.store_scatter(predv, [rlv], predvec, mask=lane0)
        plsc.store_scatter(tgenv, [rlv], tg, mask=lane0)
        plsc.store_scatter(scopv, [rlv], jnp.full((L,), scop), mask=lane0)
        return carry

    lax.fori_loop(0, RPW, row_body, 0)

    pltpu.sync_copy(predv, pred_out.at[pl.ds(base, RPW)])
    pltpu.sync_copy(tgenv, tgen_out.at[pl.ds(base, RPW)])
    pltpu.sync_copy(scopv, scop_out.at[pl.ds(base, RPW)])


_sc_call = pl.kernel(
    _sc_body,
    out_type=[
        jax.ShapeDtypeStruct((N,), jnp.int32),
        jax.ShapeDtypeStruct((N,), jnp.float32),
        jax.ShapeDtypeStruct((N,), jnp.float32),
    ],
    mesh=plsc.VectorSubcoreMesh(core_axis_name="c", subcore_axis_name="s"),
    scratch_types=[
        pltpu.VMEM((V,), jnp.float32),
        pltpu.VMEM((V,), jnp.float32),
        pltpu.VMEM((S,), jnp.float32),
        pltpu.VMEM((S,), jnp.float32),
        pltpu.VMEM((S,), jnp.int32),
        pltpu.VMEM((S + L,), jnp.int32),
        pltpu.VMEM((S,), jnp.float32),
        pltpu.VMEM((RPW,), jnp.int32),
        pltpu.VMEM((RPW,), jnp.int32),
        pltpu.VMEM((RPW,), jnp.float32),
        pltpu.VMEM((RPW,), jnp.float32),
        pltpu.SemaphoreType.DMA,
        pltpu.SemaphoreType.DMA,
    ],
    compiler_params=pltpu.CompilerParams(needs_layout_passes=False),
)


def _tc_body(pred_ref, targ_ref, tgen_ref, scop_ref, eps_ref,
             loss_ref, nw_ref, nc_ref):
    e = eps_ref[0, 0]
    targ = targ_ref[...]
    npad = targ != 0
    tot = (tgen_ref[...] + (scop_ref[...] + e)) + e
    lg = jnp.log(tot)
    loss_ref[...] = jnp.reshape(-jnp.sum(jnp.where(npad, lg, 0.0)), (1, 1))
    nw_ref[...] = jnp.reshape(jnp.sum(npad.astype(jnp.int32)), (1, 1))
    nc_ref[...] = jnp.reshape(
        jnp.sum(jnp.where(npad & (pred_ref[...] == targ), 1, 0)), (1, 1))


_tc_call = pl.pallas_call(
    _tc_body,
    out_shape=(
        jax.ShapeDtypeStruct((1, 1), jnp.float32),
        jax.ShapeDtypeStruct((1, 1), jnp.int32),
        jax.ShapeDtypeStruct((1, 1), jnp.int32),
    ),
)


@jax.jit
def kernel(probs, attn, targ, align, src, eps):
    targ_flat = targ.reshape(N)
    voc_src = jnp.where(src < V, src, 0)
    flat = (voc_src + jnp.arange(N, dtype=jnp.int32)[:, None] * V).reshape(-1)
    copies_flat = (attn * align).reshape(-1)
    skey, sval = lax.sort((flat, copies_flat), dimension=0, num_keys=1,
                          is_stable=False)
    pred, tgen, scop = _sc_call(probs, attn, align, src, targ_flat, skey, sval)
    eps2 = jnp.asarray(eps, jnp.float32).reshape(1, 1)
    loss, nw, nc = _tc_call(
        pred.reshape(16, 128), targ_flat.reshape(16, 128),
        tgen.reshape(16, 128), scop.reshape(16, 128), eps2)
    return (loss[0, 0], pred, nw[0, 0], nc[0, 0])


# speedup vs baseline: 5.5633x; 1.1750x over previous
import jax
import jax.numpy as jnp
from jax import lax
from jax.experimental import pallas as pl
from jax.experimental.pallas import tpu as pltpu
from jax.experimental.pallas import tpu_sc as plsc

N = 2048
V = 32000
S = 512
L = 16
NW = 32
RPW = N // NW
BIG = 1 << 30


def _merge_argmax(va, pa, vb, pb):
    v = jnp.maximum(va, vb)
    p = jnp.where(va > vb, pa, jnp.where(vb > va, pb, jnp.minimum(pa, pb)))
    return v, p


def _sc_a_body(probs, attn, align, src, targ,
               dmax_out, didx_out, tgen_out, scop_out, mcop_out, oov_out,
               gp_out,
               pbuf0, pbuf1, attnv, alignv, srcv, gpv,
               targv, dmaxv, didxv, tgenv, scopv, mcopv, oovv_b, sem0, sem1):
    cc = lax.axis_index("c")
    ss = lax.axis_index("s")
    wid = ss * 2 + cc
    base = wid * RPW
    iota = lax.iota(jnp.int32, L)
    zf = jnp.zeros((L,), jnp.float32)
    neg = jnp.full((L,), jnp.float32(-3.0e38))
    zi = jnp.zeros((L,), jnp.int32)
    lane0 = iota == 0

    pltpu.sync_copy(targ.at[pl.ds(base, RPW)], targv)
    pltpu.async_copy(probs.at[base], pbuf0, sem0)

    def row_body(rl, carry):
        r = base + rl
        pltpu.sync_copy(attn.at[r], attnv)
        pltpu.sync_copy(align.at[r], alignv)
        pltpu.sync_copy(src.at[r], srcv)

        def cbody(k, ca):
            csum, cmax, cidx = ca
            cp = attnv[pl.ds(k * L, L)] * alignv[pl.ds(k * L, L)]
            m = cp > cmax
            return (csum + cp, jnp.maximum(cmax, cp),
                    jnp.where(m, jnp.full((L,), k), cidx))
        csum, cmax, cidx = lax.fori_loop(0, S // L, cbody, (zf, neg, zi))
        scop = jnp.sum(csum)
        mcop = jnp.max(cmax)
        cpos = cidx * L + iota
        icop = jnp.min(jnp.where(cmax == mcop, cpos, BIG))
        oovv = plsc.load_gather(srcv, [jnp.full((L,), icop)])

        def dense_and_gp(pb):
            def dbody(i, da):
                m0, m1, m2, m3, i0, i1, i2, i3 = da
                isp = jnp.full((L,), i)
                p = pb[pl.ds(i * 64, L)]
                k = p > m0
                m0 = jnp.maximum(m0, p); i0 = jnp.where(k, isp, i0)
                p = pb[pl.ds(i * 64 + 16, L)]
                k = p > m1
                m1 = jnp.maximum(m1, p); i1 = jnp.where(k, isp, i1)
                p = pb[pl.ds(i * 64 + 32, L)]
                k = p > m2
                m2 = jnp.maximum(m2, p); i2 = jnp.where(k, isp, i2)
                p = pb[pl.ds(i * 64 + 48, L)]
                k = p > m3
                m3 = jnp.maximum(m3, p); i3 = jnp.where(k, isp, i3)
                return m0, m1, m2, m3, i0, i1, i2, i3
            m0, m1, m2, m3, i0, i1, i2, i3 = lax.fori_loop(
                0, V // 64, dbody, (neg, neg, neg, neg, zi, zi, zi, zi))
            va, pa = _merge_argmax(m0, i0 * 64 + iota, m1, i1 * 64 + 16 + iota)
            vb, pb2 = _merge_argmax(m2, i2 * 64 + 32 + iota,
                                    m3, i3 * 64 + 48 + iota)
            mv, mp = _merge_argmax(va, pa, vb, pb2)
            dmax = jnp.max(mv)
            didx = jnp.min(jnp.where(mv == dmax, mp, BIG))
            tridx = plsc.load_gather(targv, [jnp.full((L,), rl)])
            tg = plsc.load_gather(pb, [tridx])

            def gpbody(k, carry2):
                sr = srcv[pl.ds(k * L, L)]
                voc = jnp.where(sr < V, sr, 0)
                gpv[pl.ds(k * L, L)] = plsc.load_gather(pb, [voc])
                return carry2
            lax.fori_loop(0, S // L, gpbody, 0)
            return dmax, didx, tg

        even = (rl % 2) == 0

        def with_buf0(_):
            pltpu.make_async_copy(probs.at[base], pbuf0, sem0).wait()
            @pl.when(rl < RPW - 1)
            def _():
                pltpu.async_copy(probs.at[r + 1], pbuf1, sem1)
            return dense_and_gp(pbuf0)

        def with_buf1(_):
            pltpu.make_async_copy(probs.at[base], pbuf1, sem1).wait()
            @pl.when(rl < RPW - 1)
            def _():
                pltpu.async_copy(probs.at[r + 1], pbuf0, sem0)
            return dense_and_gp(pbuf1)

        dmax, didx, tg = lax.cond(even, with_buf0, with_buf1, 0)

        pltpu.sync_copy(gpv, gp_out.at[r])
        rlv = jnp.full((L,), rl)
        plsc.store_scatter(dmaxv, [rlv], jnp.full((L,), dmax), mask=lane0)
        plsc.store_scatter(didxv, [rlv], jnp.full((L,), didx), mask=lane0)
        plsc.store_scatter(tgenv, [rlv], tg, mask=lane0)
        plsc.store_scatter(scopv, [rlv], jnp.full((L,), scop), mask=lane0)
        plsc.store_scatter(mcopv, [rlv], jnp.full((L,), mcop), mask=lane0)
        plsc.store_scatter(oovv_b, [rlv], oovv, mask=lane0)
        return carry

    lax.fori_loop(0, RPW, row_body, 0)

    pltpu.sync_copy(dmaxv, dmax_out.at[pl.ds(base, RPW)])
    pltpu.sync_copy(didxv, didx_out.at[pl.ds(base, RPW)])
    pltpu.sync_copy(tgenv, tgen_out.at[pl.ds(base, RPW)])
    pltpu.sync_copy(scopv, scop_out.at[pl.ds(base, RPW)])
    pltpu.sync_copy(mcopv, mcop_out.at[pl.ds(base, RPW)])
    pltpu.sync_copy(oovv_b, oov_out.at[pl.ds(base, RPW)])


_sc_a_call = pl.kernel(
    _sc_a_body,
    out_type=[
        jax.ShapeDtypeStruct((N,), jnp.float32),
        jax.ShapeDtypeStruct((N,), jnp.int32),
        jax.ShapeDtypeStruct((N,), jnp.float32),
        jax.ShapeDtypeStruct((N,), jnp.float32),
        jax.ShapeDtypeStruct((N,), jnp.float32),
        jax.ShapeDtypeStruct((N,), jnp.int32),
        jax.ShapeDtypeStruct((N, S), jnp.float32),
    ],
    mesh=plsc.VectorSubcoreMesh(core_axis_name="c", subcore_axis_name="s"),
    scratch_types=[
        pltpu.VMEM((V,), jnp.float32),
        pltpu.VMEM((V,), jnp.float32),
        pltpu.VMEM((S,), jnp.float32),
        pltpu.VMEM((S,), jnp.float32),
        pltpu.VMEM((S,), jnp.int32),
        pltpu.VMEM((S,), jnp.float32),
        pltpu.VMEM((RPW,), jnp.int32),
        pltpu.VMEM((RPW,), jnp.float32),
        pltpu.VMEM((RPW,), jnp.int32),
        pltpu.VMEM((RPW,), jnp.float32),
        pltpu.VMEM((RPW,), jnp.float32),
        pltpu.VMEM((RPW,), jnp.float32),
        pltpu.VMEM((RPW,), jnp.int32),
        pltpu.SemaphoreType.DMA,
        pltpu.SemaphoreType.DMA,
    ],
    compiler_params=pltpu.CompilerParams(needs_layout_passes=False),
)


def _sc_b_body(src, skey, sval, gp_in, dmax_in, didx_in, mcop_in, oov_in,
               pred_out,
               tbl, srcv, gpv, skv, svv, auxf, auxi, predv):
    cc = lax.axis_index("c")
    ss = lax.axis_index("s")
    wid = ss * 2 + cc
    base = wid * RPW
    iota = lax.iota(jnp.int32, L)
    neg = jnp.full((L,), jnp.float32(-3.0e38))
    zi = jnp.zeros((L,), jnp.int32)
    lane0 = iota == 0

    pltpu.sync_copy(dmax_in.at[pl.ds(base, RPW)], auxf.at[pl.ds(0, RPW)])
    pltpu.sync_copy(mcop_in.at[pl.ds(base, RPW)], auxf.at[pl.ds(RPW, RPW)])
    pltpu.sync_copy(didx_in.at[pl.ds(base, RPW)], auxi.at[pl.ds(0, RPW)])
    pltpu.sync_copy(oov_in.at[pl.ds(base, RPW)], auxi.at[pl.ds(RPW, RPW)])
    skv[pl.ds(S, L)] = jnp.full((L,), jnp.int32(-1))

    def row_body(rl, carry):
        r = base + rl
        pltpu.sync_copy(src.at[r], srcv)
        pltpu.sync_copy(gp_in.at[r], gpv)
        pltpu.sync_copy(skey.at[pl.ds(r * S, S)], skv.at[pl.ds(0, S)])
        pltpu.sync_copy(sval.at[pl.ds(r * S, S)], svv)

        def sbody(k, carry2):
            sr = srcv[pl.ds(k * L, L)]
            voc = jnp.where(sr < V, sr, 0)
            plsc.store_scatter(tbl, [voc], gpv[pl.ds(k * L, L)])
            return carry2
        lax.fori_loop(0, S // L, sbody, 0)

        rbase = r * V

        def gbody(k, ga):
            gmax, gidx = ga
            kk = skv[pl.ds(k * L, L)]
            kn = skv[pl.ds(k * L + 1, L)]
            vv = svv[pl.ds(k * L, L)]
            slot = kk - rbase
            gp = plsc.load_gather(tbl, [slot])
            cand = jnp.where(kk != kn, gp + vv, neg)
            m = cand > gmax
            return jnp.maximum(gmax, cand), jnp.where(m, slot, gidx)
        gmax, gidx = lax.fori_loop(0, S // L, gbody, (neg, zi))
        candmax = jnp.max(gmax)
        candidx = jnp.min(jnp.where(gmax == candmax, gidx, BIG))

        rlv = jnp.full((L,), rl)
        dmax16 = plsc.load_gather(auxf, [rlv])
        mcop16 = plsc.load_gather(auxf, [rlv + RPW])
        didx16 = plsc.load_gather(auxi, [rlv])
        oov16 = plsc.load_gather(auxi, [rlv + RPW])
        dmax = jnp.max(dmax16)
        mcop = jnp.max(mcop16)
        mvoc = jnp.maximum(dmax, candmax)
        ivoc = jnp.where(candmax > dmax, jnp.full((L,), candidx), didx16)
        predvec = jnp.where(mvoc < mcop, oov16, ivoc)
        plsc.store_scatter(predv, [rlv], predvec, mask=lane0)
        return carry

    lax.fori_loop(0, RPW, row_body, 0)
    pltpu.sync_copy(predv, pred_out.at[pl.ds(base, RPW)])


_sc_b_call = pl.kernel(
    _sc_b_body,
    out_type=[
        jax.ShapeDtypeStruct((N,), jnp.int32),
    ],
    mesh=plsc.VectorSubcoreMesh(core_axis_name="c", subcore_axis_name="s"),
    scratch_types=[
        pltpu.VMEM((V,), jnp.float32),
        pltpu.VMEM((S,), jnp.int32),
        pltpu.VMEM((S,), jnp.float32),
        pltpu.VMEM((S + L,), jnp.int32),
        pltpu.VMEM((S,), jnp.float32),
        pltpu.VMEM((2 * RPW,), jnp.float32),
        pltpu.VMEM((2 * RPW,), jnp.int32),
        pltpu.VMEM((RPW,), jnp.int32),
    ],
    compiler_params=pltpu.CompilerParams(needs_layout_passes=False),
)


def _tc_body(pred_ref, targ_ref, tgen_ref, scop_ref, eps_ref,
             loss_ref, nw_ref, nc_ref):
    e = eps_ref[0, 0]
    targ = targ_ref[...]
    npad = targ != 0
    tot = (tgen_ref[...] + (scop_ref[...] + e)) + e
    lg = jnp.log(tot)
    loss_ref[...] = jnp.reshape(-jnp.sum(jnp.where(npad, lg, 0.0)), (1, 1))
    nw_ref[...] = jnp.reshape(jnp.sum(npad.astype(jnp.int32)), (1, 1))
    nc_ref[...] = jnp.reshape(
        jnp.sum(jnp.where(npad & (pred_ref[...] == targ), 1, 0)), (1, 1))


_tc_call = pl.pallas_call(
    _tc_body,
    out_shape=(
        jax.ShapeDtypeStruct((1, 1), jnp.float32),
        jax.ShapeDtypeStruct((1, 1), jnp.int32),
        jax.ShapeDtypeStruct((1, 1), jnp.int32),
    ),
)


@jax.jit
def kernel(probs, attn, targ, align, src, eps):
    targ_flat = targ.reshape(N)
    voc_src = jnp.where(src < V, src, 0)
    flat = (voc_src + jnp.arange(N, dtype=jnp.int32)[:, None] * V).reshape(-1)
    copies_flat = (attn * align).reshape(-1)
    skey, sval = lax.sort((flat, copies_flat), dimension=0, num_keys=1,
                          is_stable=False)
    dmax, didx, tgen, scop, mcop, oov, gp = _sc_a_call(
        probs, attn, align, src, targ_flat)
    pred, = _sc_b_call(src, skey, sval, gp, dmax, didx, mcop, oov)
    eps2 = jnp.asarray(eps, jnp.float32).reshape(1, 1)
    loss, nw, nc = _tc_call(
        pred.reshape(16, 128), targ_flat.reshape(16, 128),
        tgen.reshape(16, 128), scop.reshape(16, 128), eps2)
    return (loss[0, 0], pred, nw[0, 0], nc[0, 0])


# speedup vs baseline: 6.3972x; 1.1499x over previous
import jax
import jax.numpy as jnp
from jax import lax
from jax.experimental import pallas as pl
from jax.experimental.pallas import tpu as pltpu
from jax.experimental.pallas import tpu_sc as plsc

N = 2048
V = 32000
S = 512
L = 16
NW = 32
RPW = N // NW
BIG = 1 << 30


def _merge_argmax(va, pa, vb, pb):
    v = jnp.maximum(va, vb)
    p = jnp.where(va > vb, pa, jnp.where(vb > va, pb, jnp.minimum(pa, pb)))
    return v, p


def _sc_a_body(probs, attn, align, src, targ,
               dmax_out, didx_out, tgen_out, scop_out, mcop_out, oov_out,
               gp_out,
               pbuf0, pbuf1, attnv, alignv, srcv, gpv,
               targv, dmaxv, didxv, tgenv, scopv, mcopv, oovv_b, sem0, sem1):
    cc = lax.axis_index("c")
    ss = lax.axis_index("s")
    wid = ss * 2 + cc
    base = wid * RPW
    iota = lax.iota(jnp.int32, L)
    zf = jnp.zeros((L,), jnp.float32)
    neg = jnp.full((L,), jnp.float32(-3.0e38))
    zi = jnp.zeros((L,), jnp.int32)
    lane0 = iota == 0

    pltpu.sync_copy(targ.at[pl.ds(base, RPW)], targv)
    pltpu.async_copy(probs.at[base], pbuf0, sem0)

    def row_body(rl, carry):
        r = base + rl
        pltpu.sync_copy(attn.at[r], attnv)
        pltpu.sync_copy(align.at[r], alignv)
        pltpu.sync_copy(src.at[r], srcv)

        def cbody(k, ca):
            csum, cmax, cidx = ca
            cp = attnv[pl.ds(k * L, L)] * alignv[pl.ds(k * L, L)]
            m = cp > cmax
            return (csum + cp, jnp.maximum(cmax, cp),
                    jnp.where(m, jnp.full((L,), k), cidx))
        csum, cmax, cidx = lax.fori_loop(0, S // L, cbody, (zf, neg, zi))
        scop = jnp.sum(csum)
        mcop = jnp.max(cmax)
        cpos = cidx * L + iota
        icop = jnp.min(jnp.where(cmax == mcop, cpos, BIG))
        oovv = plsc.load_gather(srcv, [jnp.full((L,), icop)])

        def dense_and_gp(pb):
            def dbody(i, da):
                m0, m1, m2, m3, i0, i1, i2, i3 = da
                isp = jnp.full((L,), i)
                p = pb[pl.ds(i * 64, L)]
                k = p > m0
                m0 = jnp.maximum(m0, p); i0 = jnp.where(k, isp, i0)
                p = pb[pl.ds(i * 64 + 16, L)]
                k = p > m1
                m1 = jnp.maximum(m1, p); i1 = jnp.where(k, isp, i1)
                p = pb[pl.ds(i * 64 + 32, L)]
                k = p > m2
                m2 = jnp.maximum(m2, p); i2 = jnp.where(k, isp, i2)
                p = pb[pl.ds(i * 64 + 48, L)]
                k = p > m3
                m3 = jnp.maximum(m3, p); i3 = jnp.where(k, isp, i3)
                return m0, m1, m2, m3, i0, i1, i2, i3
            m0, m1, m2, m3, i0, i1, i2, i3 = lax.fori_loop(
                0, V // 64, dbody, (neg, neg, neg, neg, zi, zi, zi, zi))
            va, pa = _merge_argmax(m0, i0 * 64 + iota, m1, i1 * 64 + 16 + iota)
            vb, pb2 = _merge_argmax(m2, i2 * 64 + 32 + iota,
                                    m3, i3 * 64 + 48 + iota)
            mv, mp = _merge_argmax(va, pa, vb, pb2)
            dmax = jnp.max(mv)
            didx = jnp.min(jnp.where(mv == dmax, mp, BIG))
            tridx = plsc.load_gather(targv, [jnp.full((L,), rl)])
            tg = plsc.load_gather(pb, [tridx])

            def gpbody(k, carry2):
                sr = srcv[pl.ds(k * L, L)]
                voc = jnp.where(sr < V, sr, 0)
                gpv[pl.ds(k * L, L)] = plsc.load_gather(pb, [voc])
                return carry2
            lax.fori_loop(0, S // L, gpbody, 0)
            return dmax, didx, tg

        even = (rl % 2) == 0

        def with_buf0(_):
            pltpu.make_async_copy(probs.at[base], pbuf0, sem0).wait()
            @pl.when(rl < RPW - 1)
            def _():
                pltpu.async_copy(probs.at[r + 1], pbuf1, sem1)
            return dense_and_gp(pbuf0)

        def with_buf1(_):
            pltpu.make_async_copy(probs.at[base], pbuf1, sem1).wait()
            @pl.when(rl < RPW - 1)
            def _():
                pltpu.async_copy(probs.at[r + 1], pbuf0, sem0)
            return dense_and_gp(pbuf1)

        dmax, didx, tg = lax.cond(even, with_buf0, with_buf1, 0)

        pltpu.sync_copy(gpv, gp_out.at[r])
        rlv = jnp.full((L,), rl)
        plsc.store_scatter(dmaxv, [rlv], jnp.full((L,), dmax), mask=lane0)
        plsc.store_scatter(didxv, [rlv], jnp.full((L,), didx), mask=lane0)
        plsc.store_scatter(tgenv, [rlv], tg, mask=lane0)
        plsc.store_scatter(scopv, [rlv], jnp.full((L,), scop), mask=lane0)
        plsc.store_scatter(mcopv, [rlv], jnp.full((L,), mcop), mask=lane0)
        plsc.store_scatter(oovv_b, [rlv], oovv, mask=lane0)
        return carry

    lax.fori_loop(0, RPW, row_body, 0)

    pltpu.sync_copy(dmaxv, dmax_out.at[pl.ds(base, RPW)])
    pltpu.sync_copy(didxv, didx_out.at[pl.ds(base, RPW)])
    pltpu.sync_copy(tgenv, tgen_out.at[pl.ds(base, RPW)])
    pltpu.sync_copy(scopv, scop_out.at[pl.ds(base, RPW)])
    pltpu.sync_copy(mcopv, mcop_out.at[pl.ds(base, RPW)])
    pltpu.sync_copy(oovv_b, oov_out.at[pl.ds(base, RPW)])


_sc_a_call = pl.kernel(
    _sc_a_body,
    out_type=[
        jax.ShapeDtypeStruct((N,), jnp.float32),
        jax.ShapeDtypeStruct((N,), jnp.int32),
        jax.ShapeDtypeStruct((N,), jnp.float32),
        jax.ShapeDtypeStruct((N,), jnp.float32),
        jax.ShapeDtypeStruct((N,), jnp.float32),
        jax.ShapeDtypeStruct((N,), jnp.int32),
        jax.ShapeDtypeStruct((N, S), jnp.float32),
    ],
    mesh=plsc.VectorSubcoreMesh(core_axis_name="c", subcore_axis_name="s"),
    scratch_types=[
        pltpu.VMEM((V,), jnp.float32),
        pltpu.VMEM((V,), jnp.float32),
        pltpu.VMEM((S,), jnp.float32),
        pltpu.VMEM((S,), jnp.float32),
        pltpu.VMEM((S,), jnp.int32),
        pltpu.VMEM((S,), jnp.float32),
        pltpu.VMEM((RPW,), jnp.int32),
        pltpu.VMEM((RPW,), jnp.float32),
        pltpu.VMEM((RPW,), jnp.int32),
        pltpu.VMEM((RPW,), jnp.float32),
        pltpu.VMEM((RPW,), jnp.float32),
        pltpu.VMEM((RPW,), jnp.float32),
        pltpu.VMEM((RPW,), jnp.int32),
        pltpu.SemaphoreType.DMA,
        pltpu.SemaphoreType.DMA,
    ],
    compiler_params=pltpu.CompilerParams(needs_layout_passes=False),
)


def _sc_b_body(src, skey, sval, gp_in, dmax_in, didx_in, mcop_in, oov_in,
               pred_out,
               tbl, srcv0, srcv1, gpv0, gpv1, skv0, skv1, svv0, svv1,
               auxf, auxi, predv, sem0, sem1):
    cc = lax.axis_index("c")
    ss = lax.axis_index("s")
    wid = ss * 2 + cc
    base = wid * RPW
    iota = lax.iota(jnp.int32, L)
    neg = jnp.full((L,), jnp.float32(-3.0e38))
    zi = jnp.zeros((L,), jnp.int32)
    lane0 = iota == 0

    pltpu.sync_copy(dmax_in.at[pl.ds(base, RPW)], auxf.at[pl.ds(0, RPW)])
    pltpu.sync_copy(mcop_in.at[pl.ds(base, RPW)], auxf.at[pl.ds(RPW, RPW)])
    pltpu.sync_copy(didx_in.at[pl.ds(base, RPW)], auxi.at[pl.ds(0, RPW)])
    pltpu.sync_copy(oov_in.at[pl.ds(base, RPW)], auxi.at[pl.ds(RPW, RPW)])
    skv0[pl.ds(S, L)] = jnp.full((L,), jnp.int32(-1))
    skv1[pl.ds(S, L)] = jnp.full((L,), jnp.int32(-1))

    def start_load(r, srcb, gpb, skb, svb, sem):
        pltpu.async_copy(src.at[r], srcb, sem)
        pltpu.async_copy(gp_in.at[r], gpb, sem)
        pltpu.async_copy(skey.at[pl.ds(r * S, S)], skb.at[pl.ds(0, S)], sem)
        pltpu.async_copy(sval.at[pl.ds(r * S, S)], svb, sem)

    def drain(r, srcb, gpb, skb, svb, sem):
        pltpu.make_async_copy(src.at[r], srcb, sem).wait()
        pltpu.make_async_copy(gp_in.at[r], gpb, sem).wait()
        pltpu.make_async_copy(skey.at[pl.ds(r * S, S)],
                              skb.at[pl.ds(0, S)], sem).wait()
        pltpu.make_async_copy(sval.at[pl.ds(r * S, S)], svb, sem).wait()

    start_load(base, srcv0, gpv0, skv0, svv0, sem0)

    def process(r, rl, srcv, gpv, skv, svv):
        def sbody(k, carry2):
            sr = srcv[pl.ds(k * 2 * L, L)]
            voc = jnp.where(sr < V, sr, 0)
            plsc.store_scatter(tbl, [voc], gpv[pl.ds(k * 2 * L, L)])
            sr2 = srcv[pl.ds(k * 2 * L + L, L)]
            voc2 = jnp.where(sr2 < V, sr2, 0)
            plsc.store_scatter(tbl, [voc2], gpv[pl.ds(k * 2 * L + L, L)])
            return carry2
        lax.fori_loop(0, S // (2 * L), sbody, 0)

        rbase = r * V

        def half(k, ga, off):
            gmax, gidx = ga
            kk = skv[pl.ds(k * 2 * L + off, L)]
            kn = skv[pl.ds(k * 2 * L + off + 1, L)]
            vv = svv[pl.ds(k * 2 * L + off, L)]
            slot = kk - rbase
            gp = plsc.load_gather(tbl, [slot])
            cand = jnp.where(kk != kn, gp + vv, neg)
            m = cand > gmax
            return jnp.maximum(gmax, cand), jnp.where(m, slot, gidx)

        def gbody(k, ga):
            ga0, ga1 = ga
            return half(k, ga0, 0), half(k, ga1, L)
        (gmax0, gidx0), (gmax1, gidx1) = lax.fori_loop(
            0, S // (2 * L), gbody, ((neg, zi), (neg, zi)))
        gmax = jnp.maximum(gmax0, gmax1)
        gidx = jnp.where(gmax0 > gmax1, gidx0,
                         jnp.where(gmax1 > gmax0, gidx1,
                                   jnp.minimum(gidx0, gidx1)))
        candmax = jnp.max(gmax)
        candidx = jnp.min(jnp.where(gmax == candmax, gidx, BIG))

        rlv = jnp.full((L,), rl)
        dmax16 = plsc.load_gather(auxf, [rlv])
        mcop16 = plsc.load_gather(auxf, [rlv + RPW])
        didx16 = plsc.load_gather(auxi, [rlv])
        oov16 = plsc.load_gather(auxi, [rlv + RPW])
        dmax = jnp.max(dmax16)
        mcop = jnp.max(mcop16)
        mvoc = jnp.maximum(dmax, candmax)
        ivoc = jnp.where(candmax > dmax, jnp.full((L,), candidx), didx16)
        predvec = jnp.where(mvoc < mcop, oov16, ivoc)
        plsc.store_scatter(predv, [rlv], predvec, mask=lane0)

    def row_body(rl, carry):
        r = base + rl
        even = (rl % 2) == 0

        def with_buf0(_):
            drain(r, srcv0, gpv0, skv0, svv0, sem0)
            @pl.when(rl < RPW - 1)
            def _():
                start_load(r + 1, srcv1, gpv1, skv1, svv1, sem1)
            process(r, rl, srcv0, gpv0, skv0, svv0)
            return 0

        def with_buf1(_):
            drain(r, srcv1, gpv1, skv1, svv1, sem1)
            @pl.when(rl < RPW - 1)
            def _():
                start_load(r + 1, srcv0, gpv0, skv0, svv0, sem0)
            process(r, rl, srcv1, gpv1, skv1, svv1)
            return 0

        lax.cond(even, with_buf0, with_buf1, 0)
        return carry

    lax.fori_loop(0, RPW, row_body, 0)
    pltpu.sync_copy(predv, pred_out.at[pl.ds(base, RPW)])


_sc_b_call = pl.kernel(
    _sc_b_body,
    out_type=[
        jax.ShapeDtypeStruct((N,), jnp.int32),
    ],
    mesh=plsc.VectorSubcoreMesh(core_axis_name="c", subcore_axis_name="s"),
    scratch_types=[
        pltpu.VMEM((V,), jnp.float32),
        pltpu.VMEM((S,), jnp.int32),
        pltpu.VMEM((S,), jnp.int32),
        pltpu.VMEM((S,), jnp.float32),
        pltpu.VMEM((S,), jnp.float32),
        pltpu.VMEM((S + L,), jnp.int32),
        pltpu.VMEM((S + L,), jnp.int32),
        pltpu.VMEM((S,), jnp.float32),
        pltpu.VMEM((S,), jnp.float32),
        pltpu.VMEM((2 * RPW,), jnp.float32),
        pltpu.VMEM((2 * RPW,), jnp.int32),
        pltpu.VMEM((RPW,), jnp.int32),
        pltpu.SemaphoreType.DMA,
        pltpu.SemaphoreType.DMA,
    ],
    compiler_params=pltpu.CompilerParams(needs_layout_passes=False),
)


def _tc_body(pred_ref, targ_ref, tgen_ref, scop_ref, eps_ref,
             loss_ref, nw_ref, nc_ref):
    e = eps_ref[0, 0]
    targ = targ_ref[...]
    npad = targ != 0
    tot = (tgen_ref[...] + (scop_ref[...] + e)) + e
    lg = jnp.log(tot)
    loss_ref[...] = jnp.reshape(-jnp.sum(jnp.where(npad, lg, 0.0)), (1, 1))
    nw_ref[...] = jnp.reshape(jnp.sum(npad.astype(jnp.int32)), (1, 1))
    nc_ref[...] = jnp.reshape(
        jnp.sum(jnp.where(npad & (pred_ref[...] == targ), 1, 0)), (1, 1))


_tc_call = pl.pallas_call(
    _tc_body,
    out_shape=(
        jax.ShapeDtypeStruct((1, 1), jnp.float32),
        jax.ShapeDtypeStruct((1, 1), jnp.int32),
        jax.ShapeDtypeStruct((1, 1), jnp.int32),
    ),
)


@jax.jit
def kernel(probs, attn, targ, align, src, eps):
    targ_flat = targ.reshape(N)
    voc_src = jnp.where(src < V, src, 0)
    flat = (voc_src + jnp.arange(N, dtype=jnp.int32)[:, None] * V).reshape(-1)
    copies_flat = (attn * align).reshape(-1)
    skey, sval = lax.sort((flat, copies_flat), dimension=0, num_keys=1,
                          is_stable=False)
    dmax, didx, tgen, scop, mcop, oov, gp = _sc_a_call(
        probs, attn, align, src, targ_flat)
    pred, = _sc_b_call(src, skey, sval, gp, dmax, didx, mcop, oov)
    eps2 = jnp.asarray(eps, jnp.float32).reshape(1, 1)
    loss, nw, nc = _tc_call(
        pred.reshape(16, 128), targ_flat.reshape(16, 128),
        tgen.reshape(16, 128), scop.reshape(16, 128), eps2)
    return (loss[0, 0], pred, nw[0, 0], nc[0, 0])
